# Initial kernel scaffold; baseline (speedup 1.0000x reference)
import jax
import jax.numpy as jnp
from jax.experimental import pallas as pl


def kernel(x, table):
    raise NotImplementedError("write your pallas kernel here")


# speedup vs baseline: 2.4149x; 2.4149x over previous
import functools
import math

import jax
import jax.numpy as jnp
from jax import lax
from jax.experimental import pallas as pl
from jax.experimental.pallas import tpu as pltpu
from jax.experimental.pallas import tpu_sc as plsc

EMB = 128
SCALE = math.sqrt(128.0)

NC = 2
NS = 16
NW = NC * NS
CH = 128
LANES = 16


@functools.cache
def _build(B):
    assert B % (NW * CH) == 0
    nchunk = B // (NW * CH)
    b_per_w = nchunk * CH
    mesh = plsc.VectorSubcoreMesh(core_axis_name="c", subcore_axis_name="s")

    @functools.partial(
        pl.kernel,
        mesh=mesh,
        out_type=jax.ShapeDtypeStruct((B, EMB), jnp.float32),
        scratch_types=[
            pltpu.VMEM((nchunk, CH), jnp.int32),
            pltpu.VMEM((CH, EMB), jnp.float32),
            pltpu.SemaphoreType.DMA,
        ],
    )
    def emb_kernel(idx_hbm, table_hbm, out_hbm, idx_v, rows_v, sem):
        wid = lax.axis_index("s") * NC + lax.axis_index("c")
        base = wid * b_per_w
        pltpu.sync_copy(idx_hbm.at[wid], idx_v)

        def chunk_body(c, carry):
            pltpu.async_copy(table_hbm.at[idx_v.at[c]], rows_v, sem).wait()

            def row_body(r, carry2):
                for j in range(EMB // LANES):
                    sl = pl.ds(j * LANES, LANES)
                    rows_v[r, sl] = rows_v[r, sl] * SCALE
                return carry2

            lax.fori_loop(0, CH, row_body, 0)
            pltpu.sync_copy(rows_v, out_hbm.at[pl.ds(base + c * CH, CH)])
            return carry

        lax.fori_loop(0, nchunk, chunk_body, 0)

    return emb_kernel


def kernel(x, table):
    s0, s1 = x.shape
    B = s0 * s1
    idx = x.reshape(NW, B // (NW * CH), CH).astype(jnp.int32)
    out = _build(B)(idx, table)
    return out.reshape(s0, s1, EMB)


# speedup vs baseline: 2.9426x; 1.2185x over previous
import functools
import math

import jax
import jax.numpy as jnp
from jax import lax
from jax.experimental import pallas as pl
from jax.experimental.pallas import tpu as pltpu
from jax.experimental.pallas import tpu_sc as plsc

EMB = 128
SCALE = math.sqrt(128.0)

NC = 2
NS = 16
NW = NC * NS
CH = 128
LANES = 16
NBUF = 5
RU = 2


@functools.cache
def _build(B):
    assert B % (NW * CH) == 0
    nchunk = B // (NW * CH)
    assert nchunk % NBUF == 0
    b_per_w = nchunk * CH
    mesh = plsc.VectorSubcoreMesh(core_axis_name="c", subcore_axis_name="s")

    @functools.partial(
        pl.kernel,
        mesh=mesh,
        out_type=jax.ShapeDtypeStruct((B, EMB), jnp.float32),
        scratch_types=[
            pltpu.VMEM((nchunk, CH), jnp.int32),
        ]
        + [pltpu.VMEM((CH, EMB), jnp.float32) for _ in range(NBUF)]
        + [pltpu.SemaphoreType.DMA for _ in range(NBUF)],
    )
    def emb_kernel(idx_hbm, table_hbm, out_hbm, idx_v, *bufs_sems):
        bufs = bufs_sems[:NBUF]
        sems = bufs_sems[NBUF:]
        wid = lax.axis_index("s") * NC + lax.axis_index("c")
        base = wid * b_per_w
        pltpu.sync_copy(idx_hbm.at[wid], idx_v)

        for b in range(NBUF):
            pltpu.async_copy(table_hbm.at[idx_v.at[b]], bufs[b], sems[b])

        def outer_body(o, carry):
            for b in range(NBUF):
                cc = o * NBUF + b
                pltpu.make_async_copy(
                    table_hbm.at[idx_v.at[cc]], bufs[b], sems[b]
                ).wait()

                def row_body(r, carry2):
                    for rr in range(RU):
                        for j in range(EMB // LANES):
                            sl = pl.ds(j * LANES, LANES)
                            bufs[b][r * RU + rr, sl] = bufs[b][r * RU + rr, sl] * SCALE
                    return carry2

                lax.fori_loop(0, CH // RU, row_body, 0)
                pltpu.sync_copy(bufs[b], out_hbm.at[pl.ds(base + cc * CH, CH)])
                nxt = cc + NBUF

                @pl.when(nxt < nchunk)
                def _():
                    pltpu.async_copy(table_hbm.at[idx_v.at[nxt]], bufs[b], sems[b])

            return carry

        lax.fori_loop(0, nchunk // NBUF, outer_body, 0)

    return emb_kernel


def kernel(x, table):
    s0, s1 = x.shape
    B = s0 * s1
    idx = x.reshape(NW, B // (NW * CH), CH).astype(jnp.int32)
    out = _build(B)(idx, table)
    return out.reshape(s0, s1, EMB)
